# Initial kernel scaffold; baseline (speedup 1.0000x reference)
import jax
import jax.numpy as jnp
from jax.experimental import pallas as pl


def kernel(x, edge_index, edge_attr, user_features, W_gcn1, b_gcn1, W_gcn2, b_gcn2, W_fc1, b_fc1, W_fc2, b_fc2, W_final, b_final):
    raise NotImplementedError("write your pallas kernel here")


# speedup vs baseline: 7.6530x; 7.6530x over previous
import functools

import jax
import jax.numpy as jnp
from jax import lax
from jax.experimental import pallas as pl
from jax.experimental.pallas import tpu as pltpu
from jax.experimental.pallas import tpu_sc as plsc

N = 10000
E = 320000
NP = 10240
NSUB = 16
F32 = jnp.float32


def _dot_t(a, b):
    return lax.dot_general(a, b, (((1,), (1,)), ((), ())),
                           preferred_element_type=F32)


def _front_body(x_ref, uf_ref, w1_ref, wfc1_ref, bfc1_ref, wfc2_ref,
                bfc2_ref, wfl_ref, x1_ref, mlp_ref):
    x1_ref[...] = _dot_t(x_ref[...], w1_ref[...])
    h = jnp.maximum(_dot_t(uf_ref[...], wfc1_ref[...]) + bfc1_ref[...], 0.0)
    m = _dot_t(h, wfc2_ref[...]) + bfc2_ref[...]
    mlp_ref[...] = jnp.dot(m, wfl_ref[...], preferred_element_type=F32)


def _front(x, uf, W1, Wfc1, bfc1, Wfc2, bfc2, WfL_T):
    blk = 1000
    grid = (N // blk,)
    full = lambda shape: pl.BlockSpec(shape, lambda i: tuple(0 for _ in shape))
    return pl.pallas_call(
        _front_body,
        grid=grid,
        in_specs=[
            pl.BlockSpec((blk, 128), lambda i: (i, 0)),
            pl.BlockSpec((blk, 128), lambda i: (i, 0)),
            full((256, 128)), full((256, 128)), full((256,)),
            full((128, 256)), full((128,)), full((128, 1)),
        ],
        out_specs=[
            pl.BlockSpec((blk, 256), lambda i: (i, 0)),
            pl.BlockSpec((blk, 1), lambda i: (i, 0)),
        ],
        out_shape=[
            jax.ShapeDtypeStruct((N, 256), F32),
            jax.ShapeDtypeStruct((N, 1), F32),
        ],
    )(x, uf, W1, Wfc1, bfc1, Wfc2, bfc2, WfL_T)


def _scale1_body(dega_ref, degb_ref, x1_ref, dis_ref, xsa_ref, xsb_ref):
    deg = dega_ref[...] + degb_ref[...] + 1.0
    dis = lax.rsqrt(deg)
    dis_ref[...] = dis
    xs = x1_ref[...] * dis
    xsa_ref[...] = xs[:, :128]
    xsb_ref[...] = xs[:, 128:]


def _scale1(dega, degb, x1):
    blk = 1000
    return pl.pallas_call(
        _scale1_body,
        grid=(N // blk,),
        in_specs=[
            pl.BlockSpec((blk, 1), lambda i: (i, 0)),
            pl.BlockSpec((blk, 1), lambda i: (i, 0)),
            pl.BlockSpec((blk, 256), lambda i: (i, 0)),
        ],
        out_specs=[
            pl.BlockSpec((blk, 1), lambda i: (i, 0)),
            pl.BlockSpec((blk, 128), lambda i: (i, 0)),
            pl.BlockSpec((blk, 128), lambda i: (i, 0)),
        ],
        out_shape=[
            jax.ShapeDtypeStruct((N, 1), F32),
            jax.ShapeDtypeStruct((N, 128), F32),
            jax.ShapeDtypeStruct((N, 128), F32),
        ],
    )(dega, degb, x1)


def _mid_body(o1a_ref, o1b_ref, dis_ref, x1_ref, b1_ref, w2_ref,
              x2_ref, xs2_ref):
    d = dis_ref[...]
    d2 = d * d
    x1 = x1_ref[...]
    b1 = b1_ref[...]
    ga = jnp.maximum(o1a_ref[...] * d + x1[:, :128] * d2 + b1[:128], 0.0)
    gb = jnp.maximum(o1b_ref[...] * d + x1[:, 128:] * d2 + b1[128:], 0.0)
    w2 = w2_ref[...]
    x2 = _dot_t(ga, w2[:, :128]) + _dot_t(gb, w2[:, 128:])
    x2_ref[...] = x2
    xs2_ref[...] = x2 * d


def _mid(o1a, o1b, dis, x1, b1, W2):
    blk = 1000
    full = lambda shape: pl.BlockSpec(shape, lambda i: tuple(0 for _ in shape))
    return pl.pallas_call(
        _mid_body,
        grid=(N // blk,),
        in_specs=[
            pl.BlockSpec((blk, 128), lambda i: (i, 0)),
            pl.BlockSpec((blk, 128), lambda i: (i, 0)),
            pl.BlockSpec((blk, 1), lambda i: (i, 0)),
            pl.BlockSpec((blk, 256), lambda i: (i, 0)),
            full((256,)), full((128, 256)),
        ],
        out_specs=[
            pl.BlockSpec((blk, 128), lambda i: (i, 0)),
            pl.BlockSpec((blk, 128), lambda i: (i, 0)),
        ],
        out_shape=[
            jax.ShapeDtypeStruct((N, 128), F32),
            jax.ShapeDtypeStruct((N, 128), F32),
        ],
    )(o1a, o1b, dis, x1, b1, W2)


def _final_body(o2a_ref, o2b_ref, dis_ref, x2_ref, b2_ref, mlp_ref,
                wfr_ref, bf_ref, out_ref):
    d = dis_ref[...]
    d2 = d * d
    x2 = x2_ref[...]
    b2 = b2_ref[...]
    wfr = wfr_ref[...]
    g = (o2a_ref[...] + o2b_ref[...]) * d + x2 * d2 + b2
    out_ref[...] = (mlp_ref[...]
                    + jnp.dot(g, wfr, preferred_element_type=F32)
                    + bf_ref[...])


def _final(o2a, o2b, dis, x2, b2, mlp_part, WfR_T, bf):
    blk = 1000
    full = lambda shape: pl.BlockSpec(shape, lambda i: tuple(0 for _ in shape))
    return pl.pallas_call(
        _final_body,
        grid=(N // blk,),
        in_specs=[
            pl.BlockSpec((blk, 128), lambda i: (i, 0)),
            pl.BlockSpec((blk, 128), lambda i: (i, 0)),
            pl.BlockSpec((blk, 1), lambda i: (i, 0)),
            pl.BlockSpec((blk, 128), lambda i: (i, 0)),
            full((128,)),
            pl.BlockSpec((blk, 1), lambda i: (i, 0)),
            full((128, 1)), full((1, 1)),
        ],
        out_specs=pl.BlockSpec((blk, 1), lambda i: (i, 0)),
        out_shape=jax.ShapeDtypeStruct((N, 1), F32),
    )(o2a, o2b, dis, x2, b2, mlp_part, WfR_T, bf)


_SC_MESH = dict(core_axis_name="c", subcore_axis_name="s")


def _deg_sc(dst, ew):
    CH = 80
    per_tile = E // 32
    n_ch = per_tile // CH
    rpt = NP // NSUB

    @functools.partial(
        pl.kernel,
        mesh=plsc.VectorSubcoreMesh(**_SC_MESH),
        out_type=jax.ShapeDtypeStruct((2, NP), F32),
        scratch_types=[
            pltpu.VMEM_SHARED((NP,), F32),
            pltpu.VMEM((CH,), jnp.int32),
            pltpu.VMEM((CH,), F32),
            pltpu.VMEM((rpt,), F32),
        ],
    )
    def k(dst_hbm, ew_hbm, out_hbm, acc_sh, dst_v, ew_v, zero_v):
        c = lax.axis_index("c")
        s = lax.axis_index("s")
        @pl.loop(0, rpt // 16)
        def _(r):
            zero_v[pl.ds(r * 16, 16)] = jnp.zeros((16,), F32)
        pltpu.sync_copy(zero_v, acc_sh.at[pl.ds(s * rpt, rpt)])
        plsc.subcore_barrier()

        base_t = c * (E // 2) + s * per_tile

        @pl.loop(0, n_ch)
        def _(j):
            base = base_t + j * CH
            pltpu.sync_copy(dst_hbm.at[pl.ds(base, CH)], dst_v)
            pltpu.sync_copy(ew_hbm.at[pl.ds(base, CH)], ew_v)
            pltpu.sync_copy(ew_v, acc_sh.at[dst_v], add=True)

        plsc.subcore_barrier()
        pltpu.sync_copy(acc_sh.at[pl.ds(s * rpt, rpt)],
                        out_hbm.at[c, pl.ds(s * rpt, rpt)])

    return k(dst, ew)


def _spmm_sc(tab2, src, dst, ew, D, col_split):
    CH = 80
    per_tile = E // NSUB if col_split else E // (2 * NSUB)
    n_ch = per_tile // CH
    rpt = NP // NSUB
    ZR = 128

    @functools.partial(
        pl.kernel,
        mesh=plsc.VectorSubcoreMesh(**_SC_MESH),
        out_type=jax.ShapeDtypeStruct((2, NP, D), F32),
        scratch_types=[
            pltpu.VMEM_SHARED((NP, D), F32),
            pltpu.VMEM((CH,), jnp.int32),
            pltpu.VMEM((CH,), jnp.int32),
            pltpu.VMEM((CH,), F32),
            pltpu.VMEM((CH, D), F32),
            pltpu.VMEM((ZR, D), F32),
            pltpu.SemaphoreType.DMA,
        ],
    )
    def k(tab_hbm, src_hbm, dst_hbm, ew_hbm, out_hbm,
          acc_sh, src_v, dst_v, ew_v, rows_v, zero_v, sem):
        c = lax.axis_index("c")
        s = lax.axis_index("s")
        coff = jnp.full((16,), c * N, jnp.int32)

        @pl.loop(0, ZR)
        def _(r):
            for cc in range(D // 16):
                zero_v[r, pl.ds(cc * 16, 16)] = jnp.zeros((16,), F32)
        for kk in range(rpt // ZR):
            pltpu.sync_copy(zero_v, acc_sh.at[pl.ds(s * rpt + kk * ZR, ZR)])
        plsc.subcore_barrier()

        base_t = (s * per_tile) if col_split else (c * (E // 2) + s * per_tile)

        @pl.loop(0, n_ch)
        def _(j):
            base = base_t + j * CH
            pltpu.sync_copy(src_hbm.at[pl.ds(base, CH)], src_v)
            pltpu.sync_copy(dst_hbm.at[pl.ds(base, CH)], dst_v)
            pltpu.sync_copy(ew_hbm.at[pl.ds(base, CH)], ew_v)

            if col_split:
                @pl.loop(0, CH // 16)
                def _(g):
                    sl = pl.ds(g * 16, 16)
                    src_v[sl] = src_v[sl] + coff

            pltpu.async_copy(tab_hbm.at[src_v], rows_v, sem).wait()

            @pl.loop(0, CH // 16)
            def _(g):
                w16 = ew_v[pl.ds(g * 16, 16)]
                for rr in range(16):
                    w = lax.gather(
                        w16, jnp.full((16, 1), rr, jnp.int32),
                        lax.GatherDimensionNumbers(
                            offset_dims=(), collapsed_slice_dims=(0,),
                            start_index_map=(0,)),
                        (1,), mode=lax.GatherScatterMode.PROMISE_IN_BOUNDS)
                    r = g * 16 + rr
                    for cc in range(D // 16):
                        sl = (r, pl.ds(cc * 16, 16))
                        rows_v[sl] = rows_v[sl] * w

            pltpu.sync_copy(rows_v, acc_sh.at[dst_v], add=True)

        plsc.subcore_barrier()
        pltpu.sync_copy(acc_sh.at[pl.ds(s * rpt, rpt)],
                        out_hbm.at[c, pl.ds(s * rpt, rpt)])

    return k(tab2, src, dst, ew)


def kernel(x, edge_index, edge_attr, user_features,
           W_gcn1, b_gcn1, W_gcn2, b_gcn2,
           W_fc1, b_fc1, W_fc2, b_fc2,
           W_final, b_final):
    src = edge_index[0].astype(jnp.int32)
    dst = edge_index[1].astype(jnp.int32)
    ew = edge_attr.astype(F32)

    WfL_T = W_final[:, :128].T
    WfR_T = W_final[:, 128:].T
    bf = b_final.reshape(1, 1)

    x1, mlp_part = _front(x, user_features, W_gcn1, W_fc1, b_fc1,
                          W_fc2, b_fc2, WfL_T)

    degp = _deg_sc(dst, ew)
    dis, xsa, xsb = _scale1(degp[0, :N].reshape(N, 1),
                            degp[1, :N].reshape(N, 1), x1)

    o1 = _spmm_sc(jnp.concatenate([xsa, xsb], axis=0), src, dst, ew, 128, True)

    x2, xs2 = _mid(o1[0, :N], o1[1, :N], dis, x1, b_gcn1, W_gcn2)

    o2 = _spmm_sc(xs2, src, dst, ew, 128, False)

    return _final(o2[0, :N], o2[1, :N], dis, x2, b_gcn2, mlp_part, WfR_T, bf)


# speedup vs baseline: 7.9633x; 1.0406x over previous
import functools

import jax
import jax.numpy as jnp
from jax import lax
from jax.experimental import pallas as pl
from jax.experimental.pallas import tpu as pltpu
from jax.experimental.pallas import tpu_sc as plsc

N = 10000
E = 320000
EP = 327680
NP = 10240
NSUB = 16
F32 = jnp.float32


def _dot_t(a, b):
    return lax.dot_general(a, b, (((1,), (1,)), ((), ())),
                           preferred_element_type=F32)


def _front_body(x_ref, uf_ref, w1_ref, wfc1_ref, bfc1_ref, wfc2_ref,
                bfc2_ref, wfl_ref, x1_ref, mlp_ref):
    x1_ref[...] = _dot_t(x_ref[...], w1_ref[...])
    h = jnp.maximum(_dot_t(uf_ref[...], wfc1_ref[...]) + bfc1_ref[...], 0.0)
    m = _dot_t(h, wfc2_ref[...]) + bfc2_ref[...]
    mlp_ref[...] = jnp.dot(m, wfl_ref[...], preferred_element_type=F32)


def _front(x, uf, W1, Wfc1, bfc1, Wfc2, bfc2, WfL_T):
    blk = 1000
    grid = (N // blk,)
    full = lambda shape: pl.BlockSpec(shape, lambda i: tuple(0 for _ in shape))
    return pl.pallas_call(
        _front_body,
        grid=grid,
        in_specs=[
            pl.BlockSpec((blk, 128), lambda i: (i, 0)),
            pl.BlockSpec((blk, 128), lambda i: (i, 0)),
            full((256, 128)), full((256, 128)), full((256,)),
            full((128, 256)), full((128,)), full((128, 1)),
        ],
        out_specs=[
            pl.BlockSpec((blk, 256), lambda i: (i, 0)),
            pl.BlockSpec((blk, 1), lambda i: (i, 0)),
        ],
        out_shape=[
            jax.ShapeDtypeStruct((N, 256), F32),
            jax.ShapeDtypeStruct((N, 1), F32),
        ],
    )(x, uf, W1, Wfc1, bfc1, Wfc2, bfc2, WfL_T)


def _scale1_body(dega_ref, degb_ref, x1_ref, dis_ref, xsa_ref, xsb_ref):
    deg = dega_ref[...] + degb_ref[...] + 1.0
    dis = lax.rsqrt(deg)
    dis_ref[...] = dis
    xs = x1_ref[...] * dis
    xsa_ref[...] = xs[:, :128]
    xsb_ref[...] = xs[:, 128:]


def _scale1(dega, degb, x1):
    blk = 1000
    return pl.pallas_call(
        _scale1_body,
        grid=(N // blk,),
        in_specs=[
            pl.BlockSpec((blk, 1), lambda i: (i, 0)),
            pl.BlockSpec((blk, 1), lambda i: (i, 0)),
            pl.BlockSpec((blk, 256), lambda i: (i, 0)),
        ],
        out_specs=[
            pl.BlockSpec((blk, 1), lambda i: (i, 0)),
            pl.BlockSpec((blk, 128), lambda i: (i, 0)),
            pl.BlockSpec((blk, 128), lambda i: (i, 0)),
        ],
        out_shape=[
            jax.ShapeDtypeStruct((N, 1), F32),
            jax.ShapeDtypeStruct((N, 128), F32),
            jax.ShapeDtypeStruct((N, 128), F32),
        ],
    )(dega, degb, x1)


def _mid_body(o1a_ref, o1b_ref, dis_ref, x1_ref, b1_ref, w2_ref,
              x2_ref, xs2_ref):
    d = dis_ref[...]
    d2 = d * d
    x1 = x1_ref[...]
    b1 = b1_ref[...]
    ga = jnp.maximum(o1a_ref[...] * d + x1[:, :128] * d2 + b1[:128], 0.0)
    gb = jnp.maximum(o1b_ref[...] * d + x1[:, 128:] * d2 + b1[128:], 0.0)
    w2 = w2_ref[...]
    x2 = _dot_t(ga, w2[:, :128]) + _dot_t(gb, w2[:, 128:])
    x2_ref[...] = x2
    xs2_ref[...] = x2 * d


def _mid(o1a, o1b, dis, x1, b1, W2):
    blk = 1000
    full = lambda shape: pl.BlockSpec(shape, lambda i: tuple(0 for _ in shape))
    return pl.pallas_call(
        _mid_body,
        grid=(N // blk,),
        in_specs=[
            pl.BlockSpec((blk, 128), lambda i: (i, 0)),
            pl.BlockSpec((blk, 128), lambda i: (i, 0)),
            pl.BlockSpec((blk, 1), lambda i: (i, 0)),
            pl.BlockSpec((blk, 256), lambda i: (i, 0)),
            full((256,)), full((128, 256)),
        ],
        out_specs=[
            pl.BlockSpec((blk, 128), lambda i: (i, 0)),
            pl.BlockSpec((blk, 128), lambda i: (i, 0)),
        ],
        out_shape=[
            jax.ShapeDtypeStruct((N, 128), F32),
            jax.ShapeDtypeStruct((N, 128), F32),
        ],
    )(o1a, o1b, dis, x1, b1, W2)


def _final_body(o2a_ref, o2b_ref, dis_ref, x2_ref, b2_ref, mlp_ref,
                wfr_ref, bf_ref, out_ref):
    d = dis_ref[...]
    d2 = d * d
    x2 = x2_ref[...]
    b2 = b2_ref[...]
    wfr = wfr_ref[...]
    g = (o2a_ref[...] + o2b_ref[...]) * d + x2 * d2 + b2
    out_ref[...] = (mlp_ref[...]
                    + jnp.dot(g, wfr, preferred_element_type=F32)
                    + bf_ref[...])


def _final(o2a, o2b, dis, x2, b2, mlp_part, WfR_T, bf):
    blk = 1000
    full = lambda shape: pl.BlockSpec(shape, lambda i: tuple(0 for _ in shape))
    return pl.pallas_call(
        _final_body,
        grid=(N // blk,),
        in_specs=[
            pl.BlockSpec((blk, 128), lambda i: (i, 0)),
            pl.BlockSpec((blk, 128), lambda i: (i, 0)),
            pl.BlockSpec((blk, 1), lambda i: (i, 0)),
            pl.BlockSpec((blk, 128), lambda i: (i, 0)),
            full((128,)),
            pl.BlockSpec((blk, 1), lambda i: (i, 0)),
            full((128, 1)), full((1, 1)),
        ],
        out_specs=pl.BlockSpec((blk, 1), lambda i: (i, 0)),
        out_shape=jax.ShapeDtypeStruct((N, 1), F32),
    )(o2a, o2b, dis, x2, b2, mlp_part, WfR_T, bf)


_SC_MESH = dict(core_axis_name="c", subcore_axis_name="s")


def _deg_sc(dstm, ewm):
    C = EP // 128
    cpt = C // 32
    rpt = NP // NSUB

    @functools.partial(
        pl.kernel,
        mesh=plsc.VectorSubcoreMesh(**_SC_MESH),
        out_type=jax.ShapeDtypeStruct((2, NP), F32),
        scratch_types=[
            pltpu.VMEM_SHARED((NP,), F32),
            pltpu.VMEM((cpt, 128), jnp.int32),
            pltpu.VMEM((cpt, 128), F32),
            pltpu.VMEM((rpt,), F32),
            pltpu.SemaphoreType.DMA,
        ],
    )
    def k(dst_hbm, ew_hbm, out_hbm, acc_sh, dst_v, ew_v, zero_v, sem):
        c = lax.axis_index("c")
        s = lax.axis_index("s")
        r0 = c * (C // 2) + s * cpt
        pltpu.sync_copy(dst_hbm.at[pl.ds(r0, cpt)], dst_v)
        pltpu.sync_copy(ew_hbm.at[pl.ds(r0, cpt)], ew_v)

        @pl.loop(0, rpt // 16)
        def _(r):
            zero_v[pl.ds(r * 16, 16)] = jnp.zeros((16,), F32)
        pltpu.sync_copy(zero_v, acc_sh.at[pl.ds(s * rpt, rpt)])
        plsc.subcore_barrier()

        @pl.loop(0, cpt)
        def _(j):
            pltpu.async_copy(ew_v.at[j], acc_sh.at[dst_v.at[j]], sem,
                             add=True)

        @pl.loop(0, cpt)
        def _(j):
            pltpu.make_async_copy(ew_v.at[0], acc_sh.at[dst_v.at[0]],
                                  sem).wait()

        plsc.subcore_barrier()
        pltpu.sync_copy(acc_sh.at[pl.ds(s * rpt, rpt)],
                        out_hbm.at[c, pl.ds(s * rpt, rpt)])

    return k(dstm, ewm)


def _spmm_sc(tab, srcm, dstm, ewm, col_split):
    C = EP // 128
    cpt = (C // NSUB) if col_split else (C // 32)
    B = 16
    n_batches = cpt // B
    rpt = NP // NSUB

    @functools.partial(
        pl.kernel,
        mesh=plsc.VectorSubcoreMesh(**_SC_MESH),
        out_type=jax.ShapeDtypeStruct((2, NP, 128), F32),
        scratch_types=[
            pltpu.VMEM_SHARED((NP, 128), F32),
            pltpu.VMEM((B, 128), jnp.int32),
            pltpu.VMEM((B, 128), jnp.int32),
            pltpu.VMEM((B, 128), F32),
            pltpu.VMEM((128, 128), F32),
            pltpu.VMEM((128, 128), F32),
            pltpu.SemaphoreType.DMA,
            pltpu.SemaphoreType.DMA,
        ],
    )
    def k(tab_hbm, src_hbm, dst_hbm, ew_hbm, out_hbm,
          acc_sh, src_v, dst_v, ew_v, rows0, rows1, sem0, sem1):
        c = lax.axis_index("c")
        s = lax.axis_index("s")
        r0 = (s * cpt) if col_split else (c * (C // 2) + s * cpt)

        @pl.loop(0, 128)
        def _(r):
            for cc in range(8):
                rows0[r, pl.ds(cc * 16, 16)] = jnp.zeros((16,), F32)
        for kk in range(rpt // 128):
            pltpu.sync_copy(rows0, acc_sh.at[pl.ds(s * rpt + kk * 128, 128)])

        def start_g(j, rows):
            pltpu.async_copy(tab_hbm.at[src_v.at[j]], rows,
                             sem0 if rows is rows0 else sem1)

        def wait_g(j, rows):
            pltpu.make_async_copy(tab_hbm.at[src_v.at[j]], rows,
                                  sem0 if rows is rows0 else sem1).wait()

        def mul(j, rows):
            for g in range(8):
                w16 = ew_v[j, pl.ds(g * 16, 16)]

                @pl.loop(0, 16)
                def _(rr):
                    w = lax.gather(
                        w16, jnp.full((16, 1), rr, jnp.int32),
                        lax.GatherDimensionNumbers(
                            offset_dims=(), collapsed_slice_dims=(0,),
                            start_index_map=(0,)),
                        (1,), mode=lax.GatherScatterMode.PROMISE_IN_BOUNDS)
                    r = g * 16 + rr
                    for cc in range(8):
                        sl = (r, pl.ds(cc * 16, 16))
                        rows[sl] = rows[sl] * w

        def scat(j, rows):
            pltpu.sync_copy(rows, acc_sh.at[dst_v.at[j]], add=True)

        plsc.subcore_barrier()

        @pl.loop(0, n_batches)
        def _(b):
            rb = r0 + b * B
            if col_split:
                pltpu.sync_copy(src_hbm.at[c, pl.ds(rb, B)], src_v)
            else:
                pltpu.sync_copy(src_hbm.at[pl.ds(rb, B)], src_v)
            pltpu.sync_copy(dst_hbm.at[pl.ds(rb, B)], dst_v)
            pltpu.sync_copy(ew_hbm.at[pl.ds(rb, B)], ew_v)

            start_g(0, rows0)
            start_g(1, rows1)
            for pp in range(B // 2 - 1):
                j0 = 2 * pp
                wait_g(j0, rows0)
                mul(j0, rows0)
                scat(j0, rows0)
                start_g(j0 + 2, rows0)
                wait_g(j0 + 1, rows1)
                mul(j0 + 1, rows1)
                scat(j0 + 1, rows1)
                start_g(j0 + 3, rows1)
            wait_g(B - 2, rows0)
            mul(B - 2, rows0)
            scat(B - 2, rows0)
            wait_g(B - 1, rows1)
            mul(B - 1, rows1)
            scat(B - 1, rows1)

        plsc.subcore_barrier()
        pltpu.sync_copy(acc_sh.at[pl.ds(s * rpt, rpt)],
                        out_hbm.at[c, pl.ds(s * rpt, rpt)])

    return k(tab, srcm, dstm, ewm)


def kernel(x, edge_index, edge_attr, user_features,
           W_gcn1, b_gcn1, W_gcn2, b_gcn2,
           W_fc1, b_fc1, W_fc2, b_fc2,
           W_final, b_final):
    src = edge_index[0].astype(jnp.int32)
    dst = edge_index[1].astype(jnp.int32)
    ew = edge_attr.astype(F32)

    pad = EP - E
    srcp = jnp.concatenate([src, jnp.zeros((pad,), jnp.int32)])
    dstp = jnp.concatenate([dst, jnp.full((pad,), N, jnp.int32)])
    ewp = jnp.concatenate([ew, jnp.zeros((pad,), F32)])
    srcm2 = jnp.stack([srcp, srcp + N]).reshape(2, EP // 128, 128)
    srcm = srcp.reshape(EP // 128, 128)
    dstm = dstp.reshape(EP // 128, 128)
    ewm = ewp.reshape(EP // 128, 128)

    WfL_T = W_final[:, :128].T
    WfR_T = W_final[:, 128:].T
    bf = b_final.reshape(1, 1)

    x1, mlp_part = _front(x, user_features, W_gcn1, W_fc1, b_fc1,
                          W_fc2, b_fc2, WfL_T)

    degp = _deg_sc(dstm, ewm)
    dis, xsa, xsb = _scale1(degp[0, :N].reshape(N, 1),
                            degp[1, :N].reshape(N, 1), x1)

    o1 = _spmm_sc(jnp.concatenate([xsa, xsb], axis=0), srcm2, dstm, ewm, True)

    x2, xs2 = _mid(o1[0, :N], o1[1, :N], dis, x1, b_gcn1, W_gcn2)

    o2 = _spmm_sc(xs2, srcm, dstm, ewm, False)

    return _final(o2[0, :N], o2[1, :N], dis, x2, b_gcn2, mlp_part, WfR_T, bf)


# speedup vs baseline: 9.1011x; 1.1429x over previous
import functools

import jax
import jax.numpy as jnp
from jax import lax
from jax.experimental import pallas as pl
from jax.experimental.pallas import tpu as pltpu
from jax.experimental.pallas import tpu_sc as plsc

N = 10000
E = 320000
EP = 327680
NP = 10240
NSUB = 16
F32 = jnp.float32


def _dot_t(a, b):
    return lax.dot_general(a, b, (((1,), (1,)), ((), ())),
                           preferred_element_type=F32)


def _front_body(x_ref, uf_ref, w1_ref, wfc1_ref, bfc1_ref, wfc2_ref,
                bfc2_ref, wfl_ref, x1_ref, mlp_ref):
    x1_ref[...] = _dot_t(x_ref[...], w1_ref[...])
    h = jnp.maximum(_dot_t(uf_ref[...], wfc1_ref[...]) + bfc1_ref[...], 0.0)
    m = _dot_t(h, wfc2_ref[...]) + bfc2_ref[...]
    mlp_ref[...] = jnp.dot(m, wfl_ref[...], preferred_element_type=F32)


def _front(x, uf, W1, Wfc1, bfc1, Wfc2, bfc2, WfL_T):
    blk = 1000
    grid = (N // blk,)
    full = lambda shape: pl.BlockSpec(shape, lambda i: tuple(0 for _ in shape))
    return pl.pallas_call(
        _front_body,
        grid=grid,
        in_specs=[
            pl.BlockSpec((blk, 128), lambda i: (i, 0)),
            pl.BlockSpec((blk, 128), lambda i: (i, 0)),
            full((256, 128)), full((256, 128)), full((256,)),
            full((128, 256)), full((128,)), full((128, 1)),
        ],
        out_specs=[
            pl.BlockSpec((blk, 256), lambda i: (i, 0)),
            pl.BlockSpec((blk, 1), lambda i: (i, 0)),
        ],
        out_shape=[
            jax.ShapeDtypeStruct((N, 256), F32),
            jax.ShapeDtypeStruct((N, 1), F32),
        ],
    )(x, uf, W1, Wfc1, bfc1, Wfc2, bfc2, WfL_T)


def _scale1_body(dega_ref, degb_ref, x1_ref, dis_ref, xsa_ref, xsb_ref):
    deg = dega_ref[...] + degb_ref[...] + 1.0
    dis = lax.rsqrt(deg)
    dis_ref[...] = dis
    xs = x1_ref[...] * dis
    xsa_ref[...] = xs[:, :128]
    xsb_ref[...] = xs[:, 128:]


def _scale1(dega, degb, x1):
    blk = 1000
    return pl.pallas_call(
        _scale1_body,
        grid=(N // blk,),
        in_specs=[
            pl.BlockSpec((blk, 1), lambda i: (i, 0)),
            pl.BlockSpec((blk, 1), lambda i: (i, 0)),
            pl.BlockSpec((blk, 256), lambda i: (i, 0)),
        ],
        out_specs=[
            pl.BlockSpec((blk, 1), lambda i: (i, 0)),
            pl.BlockSpec((blk, 128), lambda i: (i, 0)),
            pl.BlockSpec((blk, 128), lambda i: (i, 0)),
        ],
        out_shape=[
            jax.ShapeDtypeStruct((N, 1), F32),
            jax.ShapeDtypeStruct((N, 128), F32),
            jax.ShapeDtypeStruct((N, 128), F32),
        ],
    )(dega, degb, x1)


def _mid_body(o1a_ref, o1b_ref, dis_ref, x1_ref, b1_ref, w2_ref,
              x2_ref, xs2_ref):
    d = dis_ref[...]
    d2 = d * d
    x1 = x1_ref[...]
    b1 = b1_ref[...]
    ga = jnp.maximum(o1a_ref[...] * d + x1[:, :128] * d2 + b1[:128], 0.0)
    gb = jnp.maximum(o1b_ref[...] * d + x1[:, 128:] * d2 + b1[128:], 0.0)
    w2 = w2_ref[...]
    x2 = _dot_t(ga, w2[:, :128]) + _dot_t(gb, w2[:, 128:])
    x2_ref[...] = x2
    xs2_ref[...] = x2 * d


def _mid(o1a, o1b, dis, x1, b1, W2):
    blk = 1000
    full = lambda shape: pl.BlockSpec(shape, lambda i: tuple(0 for _ in shape))
    return pl.pallas_call(
        _mid_body,
        grid=(N // blk,),
        in_specs=[
            pl.BlockSpec((blk, 128), lambda i: (i, 0)),
            pl.BlockSpec((blk, 128), lambda i: (i, 0)),
            pl.BlockSpec((blk, 1), lambda i: (i, 0)),
            pl.BlockSpec((blk, 256), lambda i: (i, 0)),
            full((256,)), full((128, 256)),
        ],
        out_specs=[
            pl.BlockSpec((blk, 128), lambda i: (i, 0)),
            pl.BlockSpec((blk, 128), lambda i: (i, 0)),
        ],
        out_shape=[
            jax.ShapeDtypeStruct((N, 128), F32),
            jax.ShapeDtypeStruct((N, 128), F32),
        ],
    )(o1a, o1b, dis, x1, b1, W2)


def _final_body(o2a_ref, o2b_ref, dis_ref, x2_ref, b2_ref, mlp_ref,
                wfr_ref, bf_ref, out_ref):
    d = dis_ref[...]
    d2 = d * d
    x2 = x2_ref[...]
    b2 = b2_ref[...]
    wfr = wfr_ref[...]
    g = (o2a_ref[...] + o2b_ref[...]) * d + x2 * d2 + b2
    out_ref[...] = (mlp_ref[...]
                    + jnp.dot(g, wfr, preferred_element_type=F32)
                    + bf_ref[...])


def _final(o2a, o2b, dis, x2, b2, mlp_part, WfR_T, bf):
    blk = 1000
    full = lambda shape: pl.BlockSpec(shape, lambda i: tuple(0 for _ in shape))
    return pl.pallas_call(
        _final_body,
        grid=(N // blk,),
        in_specs=[
            pl.BlockSpec((blk, 128), lambda i: (i, 0)),
            pl.BlockSpec((blk, 128), lambda i: (i, 0)),
            pl.BlockSpec((blk, 1), lambda i: (i, 0)),
            pl.BlockSpec((blk, 128), lambda i: (i, 0)),
            full((128,)),
            pl.BlockSpec((blk, 1), lambda i: (i, 0)),
            full((128, 1)), full((1, 1)),
        ],
        out_specs=pl.BlockSpec((blk, 1), lambda i: (i, 0)),
        out_shape=jax.ShapeDtypeStruct((N, 1), F32),
    )(o2a, o2b, dis, x2, b2, mlp_part, WfR_T, bf)


_SC_MESH = dict(core_axis_name="c", subcore_axis_name="s")


def _deg_sc(dstm, ewm):
    C = EP // 128
    cpt = C // 32
    rpt = NP // NSUB

    @functools.partial(
        pl.kernel,
        mesh=plsc.VectorSubcoreMesh(**_SC_MESH),
        out_type=jax.ShapeDtypeStruct((2, NP), F32),
        scratch_types=[
            pltpu.VMEM_SHARED((NP,), F32),
            pltpu.VMEM((cpt, 128), jnp.int32),
            pltpu.VMEM((cpt, 128), F32),
            pltpu.VMEM((rpt,), F32),
            pltpu.SemaphoreType.DMA,
        ],
    )
    def k(dst_hbm, ew_hbm, out_hbm, acc_sh, dst_v, ew_v, zero_v, sem):
        c = lax.axis_index("c")
        s = lax.axis_index("s")
        r0 = c * (C // 2) + s * cpt
        pltpu.sync_copy(dst_hbm.at[pl.ds(r0, cpt)], dst_v)
        pltpu.sync_copy(ew_hbm.at[pl.ds(r0, cpt)], ew_v)

        @pl.loop(0, rpt // 16)
        def _(r):
            zero_v[pl.ds(r * 16, 16)] = jnp.zeros((16,), F32)
        pltpu.sync_copy(zero_v, acc_sh.at[pl.ds(s * rpt, rpt)])
        plsc.subcore_barrier()

        @pl.loop(0, cpt)
        def _(j):
            pltpu.async_copy(ew_v.at[j], acc_sh.at[dst_v.at[j]], sem,
                             add=True)

        @pl.loop(0, cpt)
        def _(j):
            pltpu.make_async_copy(ew_v.at[0], acc_sh.at[dst_v.at[0]],
                                  sem).wait()

        plsc.subcore_barrier()
        pltpu.sync_copy(acc_sh.at[pl.ds(s * rpt, rpt)],
                        out_hbm.at[c, pl.ds(s * rpt, rpt)])

    return k(dstm, ewm)


def _spmm_sc(tab, srcm, dstm, ewm, col_split):
    CH = 64
    C = EP // CH
    cpt = (C // NSUB) if col_split else (C // 32)
    B = 32
    n_batches = cpt // B
    S = 4
    L = 2
    rpt = NP // NSUB

    @functools.partial(
        pl.kernel,
        mesh=plsc.VectorSubcoreMesh(**_SC_MESH),
        out_type=jax.ShapeDtypeStruct((2, NP, 128), F32),
        scratch_types=[
            pltpu.VMEM_SHARED((NP, 128), F32),
            pltpu.VMEM((B, CH), jnp.int32),
            pltpu.VMEM((B, CH), jnp.int32),
            pltpu.VMEM((B, CH), F32),
        ] + [pltpu.VMEM((CH, 128), F32) for _ in range(S)]
          + [pltpu.SemaphoreType.DMA for _ in range(2 * S)],
    )
    def k(tab_hbm, src_hbm, dst_hbm, ew_hbm, out_hbm,
          acc_sh, src_v, dst_v, ew_v, *slots_and_sems):
        slots = slots_and_sems[:S]
        gsem = slots_and_sems[S:2 * S]
        ssem = slots_and_sems[2 * S:]
        c = lax.axis_index("c")
        s = lax.axis_index("s")
        r0 = (s * cpt) if col_split else (c * (C // 2) + s * cpt)

        z0 = slots[0]

        @pl.loop(0, CH)
        def _(r):
            for cc in range(8):
                z0[r, pl.ds(cc * 16, 16)] = jnp.zeros((16,), F32)
        for kk in range(rpt // CH):
            pltpu.sync_copy(z0, acc_sh.at[pl.ds(s * rpt + kk * CH, CH)])
        plsc.subcore_barrier()

        def start_g(j, sl):
            pltpu.async_copy(tab_hbm.at[src_v.at[j]], slots[sl], gsem[sl])

        def wait_g(j, sl):
            pltpu.make_async_copy(tab_hbm.at[src_v.at[j]], slots[sl],
                                  gsem[sl]).wait()

        def mul(j, sl):
            rows = slots[sl]
            for g in range(CH // 16):
                w16 = ew_v[j, pl.ds(g * 16, 16)]

                @plsc.parallel_loop(0, 16, unroll=2)
                def _(rr):
                    w = lax.gather(
                        w16, jnp.full((16, 1), rr, jnp.int32),
                        lax.GatherDimensionNumbers(
                            offset_dims=(), collapsed_slice_dims=(0,),
                            start_index_map=(0,)),
                        (1,), mode=lax.GatherScatterMode.PROMISE_IN_BOUNDS)
                    r = g * 16 + rr
                    for cc in range(8):
                        sl2 = (r, pl.ds(cc * 16, 16))
                        rows[sl2] = rows[sl2] * w

        def scat(j, sl):
            pltpu.async_copy(slots[sl], acc_sh.at[dst_v.at[j]], ssem[sl],
                             add=True)

        def wait_s(j, sl):
            pltpu.make_async_copy(slots[sl], acc_sh.at[dst_v.at[j]],
                                  ssem[sl]).wait()

        @pl.loop(0, n_batches)
        def _(b):
            rb = r0 + b * B
            if col_split:
                pltpu.sync_copy(src_hbm.at[c, pl.ds(rb, B)], src_v)
            else:
                pltpu.sync_copy(src_hbm.at[pl.ds(rb, B)], src_v)
            pltpu.sync_copy(dst_hbm.at[pl.ds(rb, B)], dst_v)
            pltpu.sync_copy(ew_hbm.at[pl.ds(rb, B)], ew_v)

            for j in range(L):
                start_g(j, j % S)
            for j in range(B):
                wait_g(j, j % S)
                mul(j, j % S)
                scat(j, j % S)
                if j + L < B:
                    if j + L >= S:
                        wait_s(j + L - S, (j + L) % S)
                    start_g(j + L, (j + L) % S)
            for j in range(B - S, B):
                wait_s(j, j % S)

        plsc.subcore_barrier()
        pltpu.sync_copy(acc_sh.at[pl.ds(s * rpt, rpt)],
                        out_hbm.at[c, pl.ds(s * rpt, rpt)])

    return k(tab, srcm, dstm, ewm)


def kernel(x, edge_index, edge_attr, user_features,
           W_gcn1, b_gcn1, W_gcn2, b_gcn2,
           W_fc1, b_fc1, W_fc2, b_fc2,
           W_final, b_final):
    src = edge_index[0].astype(jnp.int32)
    dst = edge_index[1].astype(jnp.int32)
    ew = edge_attr.astype(F32)

    pad = EP - E
    srcp = jnp.concatenate([src, jnp.zeros((pad,), jnp.int32)])
    dstp = jnp.concatenate([dst, jnp.full((pad,), N, jnp.int32)])
    ewp = jnp.concatenate([ew, jnp.zeros((pad,), F32)])
    srcm2 = jnp.stack([srcp, srcp + N]).reshape(2, EP // 64, 64)
    srcm = srcp.reshape(EP // 64, 64)
    dstm = dstp.reshape(EP // 64, 64)
    ewm = ewp.reshape(EP // 64, 64)
    dstm128 = dstp.reshape(EP // 128, 128)
    ewm128 = ewp.reshape(EP // 128, 128)

    WfL_T = W_final[:, :128].T
    WfR_T = W_final[:, 128:].T
    bf = b_final.reshape(1, 1)

    x1, mlp_part = _front(x, user_features, W_gcn1, W_fc1, b_fc1,
                          W_fc2, b_fc2, WfL_T)

    degp = _deg_sc(dstm128, ewm128)
    dis, xsa, xsb = _scale1(degp[0, :N].reshape(N, 1),
                            degp[1, :N].reshape(N, 1), x1)

    o1 = _spmm_sc(jnp.concatenate([xsa, xsb], axis=0), srcm2, dstm, ewm, True)

    x2, xs2 = _mid(o1[0, :N], o1[1, :N], dis, x1, b_gcn1, W_gcn2)

    o2 = _spmm_sc(xs2, srcm, dstm, ewm, False)

    return _final(o2[0, :N], o2[1, :N], dis, x2, b_gcn2, mlp_part, WfR_T, bf)
